# baseline (speedup 1.0000x reference)
import math

import jax
import jax.numpy as jnp
from jax import lax
from jax.experimental import pallas as pl
from jax.experimental.pallas import tpu as pltpu
from jax.experimental.pallas import tpu_sc as plsc

HID = 32
NW = 32
SCG = 2048
SCC = 4096
NBLK = 512
EBLK = 7168


def _wid():
    return lax.axis_index("s") * 2 + lax.axis_index("c")


def _pack_gd(g, d):
    gu = lax.bitcast_convert_type(g.astype(jnp.bfloat16),
                                  jnp.uint16).astype(jnp.int32)
    du = lax.bitcast_convert_type(d.astype(jnp.bfloat16),
                                  jnp.uint16).astype(jnp.int32)
    return jnp.bitwise_or(gu, jnp.left_shift(du, 16))


def _tables_body(post, wg1t, wd1t, w2pt, b1a, b2a, pk1, p2, d2):
    pt = post[...]
    gv = jax.lax.dot_general(wg1t[...], pt, (((1,), (0,)), ((), ())),
                             preferred_element_type=jnp.float32) + b1a[...]
    dv = jax.lax.dot_general(wd1t[...], pt, (((1,), (0,)), ((), ())),
                             preferred_element_type=jnp.float32)
    d2v = jax.lax.dot_general(w2pt[...], pt, (((1,), (0,)), ((), ())),
                              preferred_element_type=jnp.float32)
    pk1[...] = _pack_gd(gv, dv)
    p2[...] = d2v + b2a[...]
    d2[...] = d2v


def _tc_tables(post, wg1t, wd1t, w2pt, b1a, b2a):
    np_ = post.shape[1]
    grid = np_ // NBLK
    return pl.pallas_call(
        _tables_body,
        grid=(grid,),
        in_specs=[
            pl.BlockSpec((3, NBLK), lambda i: (0, i)),
            pl.BlockSpec((HID, 3), lambda i: (0, 0)),
            pl.BlockSpec((HID, 3), lambda i: (0, 0)),
            pl.BlockSpec((HID, 3), lambda i: (0, 0)),
            pl.BlockSpec((HID, 1), lambda i: (0, 0)),
            pl.BlockSpec((HID, 1), lambda i: (0, 0)),
        ],
        out_specs=[
            pl.BlockSpec((HID, NBLK), lambda i: (0, i)),
            pl.BlockSpec((HID, NBLK), lambda i: (0, i)),
            pl.BlockSpec((HID, NBLK), lambda i: (0, i)),
        ],
        out_shape=[
            jax.ShapeDtypeStruct((HID, np_), jnp.int32),
            jax.ShapeDtypeStruct((HID, np_), jnp.float32),
            jax.ShapeDtypeStruct((HID, np_), jnp.float32),
        ],
    )(post, wg1t, wd1t, w2pt, b1a, b2a)


def _gather_body(src_hbm, dst_hbm, pk_hbm, u_hbm,
                 table, sa, da, sb, db, ua, ub,
                 sema, semb, usema, usemb):
    ep = src_hbm.shape[0]
    np_ = table.shape[0]
    feat = _wid()
    pltpu.sync_copy(pk_hbm.at[pl.ds(feat * np_, np_)], table)
    nc = ep // SCG
    mask_hi = jnp.int32(-65536)

    def compute(sv, dv, uv):
        def grp(i, c):
            s = sv[pl.ds(i * 16, 16)]
            d = dv[pl.ds(i * 16, 16)]
            ws = plsc.load_gather(table, [s])
            wd = plsc.load_gather(table, [d])
            g = lax.bitcast_convert_type(jnp.left_shift(ws, 16),
                                         jnp.float32)
            dd = lax.bitcast_convert_type(jnp.bitwise_and(wd, mask_hi),
                                          jnp.float32)
            uv[pl.ds(i * 16, 16)] = jnp.maximum(g - dd, 0.0)
            return c

        lax.fori_loop(0, SCG // 16, grp, 0, unroll=4)

    def issue(ci, sv, dv, sem):
        pltpu.async_copy(src_hbm.at[pl.ds(ci * SCG, SCG)], sv, sem)
        pltpu.async_copy(dst_hbm.at[pl.ds(ci * SCG, SCG)], dv, sem)

    def wait_in(sv, dv, sem):
        pltpu.make_async_copy(src_hbm.at[pl.ds(0, SCG)], sv, sem).wait()
        pltpu.make_async_copy(dst_hbm.at[pl.ds(0, SCG)], dv, sem).wait()

    def drain_u(uv, sem):
        pltpu.make_async_copy(uv, u_hbm.at[pl.ds(0, SCG)], sem).wait()

    issue(0, sa, da, sema)

    def pair(k2, carry):
        c0 = 2 * k2
        issue(c0 + 1, sb, db, semb)
        wait_in(sa, da, sema)

        @pl.when(k2 > 0)
        def _():
            drain_u(ua, usema)

        compute(sa, da, ua)
        pltpu.async_copy(ua, u_hbm.at[pl.ds(feat * ep + c0 * SCG, SCG)],
                         usema)

        @pl.when(c0 + 2 < nc)
        def _():
            issue(c0 + 2, sa, da, sema)

        wait_in(sb, db, semb)

        @pl.when(k2 > 0)
        def _():
            drain_u(ub, usemb)

        compute(sb, db, ub)
        pltpu.async_copy(ub, u_hbm.at[pl.ds(feat * ep + (c0 + 1) * SCG,
                                            SCG)], usemb)
        return carry

    lax.fori_loop(0, nc // 2, pair, 0)
    drain_u(ua, usema)
    drain_u(ub, usemb)


def _sc_gather(src, dst, pk_flat, ep):
    mesh = plsc.VectorSubcoreMesh(core_axis_name="c", subcore_axis_name="s")
    f = pl.kernel(
        _gather_body,
        out_type=jax.ShapeDtypeStruct((HID * ep,), jnp.float32),
        mesh=mesh,
        compiler_params=pltpu.CompilerParams(needs_layout_passes=False),
        scratch_types=[
            pltpu.VMEM((pk_flat.shape[0] // HID,), jnp.int32),
            pltpu.VMEM((SCG,), jnp.int32),
            pltpu.VMEM((SCG,), jnp.int32),
            pltpu.VMEM((SCG,), jnp.int32),
            pltpu.VMEM((SCG,), jnp.int32),
            pltpu.VMEM((SCG,), jnp.float32),
            pltpu.VMEM((SCG,), jnp.float32),
            pltpu.SemaphoreType.DMA,
            pltpu.SemaphoreType.DMA,
            pltpu.SemaphoreType.DMA,
            pltpu.SemaphoreType.DMA,
        ],
    )
    return f(src, dst, pk_flat)


def _matmul_t_body(u, wb, bbt, mt):
    mt[...] = jax.lax.dot_general(
        wb[...], u[...], (((0,), (0,)), ((), ())),
        preferred_element_type=jnp.float32) + bbt[...]


def _tc_matmul_t(ut, wb, bbt):
    e = ut.shape[1]
    grid = e // EBLK
    return pl.pallas_call(
        _matmul_t_body,
        grid=(grid,),
        in_specs=[
            pl.BlockSpec((HID, EBLK), lambda i: (0, i)),
            pl.BlockSpec((HID, HID), lambda i: (0, 0)),
            pl.BlockSpec((HID, 1), lambda i: (0, 0)),
        ],
        out_specs=pl.BlockSpec((HID, EBLK), lambda i: (0, i)),
        out_shape=jax.ShapeDtypeStruct((HID, e), jnp.float32),
    )(ut, wb, bbt)


def _scatter_body(dst_hbm, mt_hbm, out_hbm, dsta, ma, dstb, mb,
                  sema, semb, table):
    ep = dst_hbm.shape[0]
    np_ = table.shape[0]
    feat = _wid()
    nrows = SCC // 128
    nc = ep // SCC

    zeros = jnp.zeros((16,), jnp.float32)

    def zbody(i, carry):
        table[pl.ds(i * 16, 16)] = zeros
        return carry

    lax.fori_loop(0, np_ // 16, zbody, 0, unroll=8)

    def compute(dst_v, m_v):
        def grp(r, c2):
            unsat_any = jnp.zeros((16,), jnp.bool_)
            for h in range(8):
                d = dst_v[pl.ds(r * 128 + h * 16, 16)]
                v = m_v[r, pl.ds(h * 16, 16)]
                cur = plsc.load_gather(table, [d])
                plsc.store_scatter(table, [d], jnp.maximum(cur, v))
                chk = plsc.load_gather(table, [d])
                unsat_any = jnp.logical_or(unsat_any, chk < v)

            @pl.when(jnp.any(unsat_any))
            def _retry():
                for h in range(8):
                    d = dst_v[pl.ds(r * 128 + h * 16, 16)]
                    v = m_v[r, pl.ds(h * 16, 16)]
                    chk = plsc.load_gather(table, [d])

                    def cond(mask):
                        return jnp.any(mask)

                    def body(mask):
                        cur2 = plsc.load_gather(table, [d], mask=mask)
                        plsc.store_scatter(table, [d],
                                           jnp.maximum(cur2, v), mask=mask)
                        chk2 = plsc.load_gather(table, [d], mask=mask)
                        return jnp.logical_and(mask, chk2 < v)

                    lax.while_loop(cond, body, chk < v)

            return c2

        lax.fori_loop(0, nrows, grp, 0)

    def issue(ci, dst_v, m_v, sem):
        pltpu.async_copy(dst_hbm.at[pl.ds(ci * SCC, SCC)], dst_v, sem)
        pltpu.async_copy(mt_hbm.at[feat, pl.ds(ci * nrows, nrows)], m_v,
                         sem)

    def wait_in(dst_v, m_v, sem):
        pltpu.make_async_copy(dst_hbm.at[pl.ds(0, SCC)], dst_v, sem).wait()
        pltpu.make_async_copy(mt_hbm.at[feat, pl.ds(0, nrows)], m_v,
                              sem).wait()

    issue(0, dsta, ma, sema)

    def pair(k2, carry):
        c0 = 2 * k2
        issue(c0 + 1, dstb, mb, semb)
        wait_in(dsta, ma, sema)
        compute(dsta, ma)

        @pl.when(c0 + 2 < nc)
        def _():
            issue(c0 + 2, dsta, ma, sema)

        wait_in(dstb, mb, semb)
        compute(dstb, mb)
        return carry

    lax.fori_loop(0, nc // 2, pair, 0)
    pltpu.sync_copy(table, out_hbm.at[pl.ds(feat * np_, np_)])


def _sc_scatter_max(dst_pad, mt3, np_):
    mesh = plsc.VectorSubcoreMesh(core_axis_name="c", subcore_axis_name="s")
    f = pl.kernel(
        _scatter_body,
        out_type=jax.ShapeDtypeStruct((HID * np_,), jnp.float32),
        mesh=mesh,
        compiler_params=pltpu.CompilerParams(needs_layout_passes=False),
        scratch_types=[
            pltpu.VMEM((SCC,), jnp.int32),
            pltpu.VMEM((SCC // 128, 128), jnp.float32),
            pltpu.VMEM((SCC,), jnp.int32),
            pltpu.VMEM((SCC // 128, 128), jnp.float32),
            pltpu.SemaphoreType.DMA,
            pltpu.SemaphoreType.DMA,
            pltpu.VMEM((np_,), jnp.float32),
        ],
    )
    return f(dst_pad, mt3)


def _g2_body(h1t, w2h, p2, d2, pk2):
    g2v = jax.lax.dot_general(
        w2h[...], h1t[...], (((0,), (0,)), ((), ())),
        preferred_element_type=jnp.float32) + p2[...]
    pk2[...] = _pack_gd(g2v, d2[...])


def _tc_g2(h1t, w2h, p2, d2):
    np_ = h1t.shape[1]
    grid = np_ // NBLK
    return pl.pallas_call(
        _g2_body,
        grid=(grid,),
        in_specs=[
            pl.BlockSpec((HID, NBLK), lambda i: (0, i)),
            pl.BlockSpec((HID, HID), lambda i: (0, 0)),
            pl.BlockSpec((HID, NBLK), lambda i: (0, i)),
            pl.BlockSpec((HID, NBLK), lambda i: (0, i)),
        ],
        out_specs=pl.BlockSpec((HID, NBLK), lambda i: (0, i)),
        out_shape=jax.ShapeDtypeStruct((HID, np_), jnp.int32),
    )(h1t, w2h, p2, d2)


def _transpose_body(ht, eye, out):
    out[...] = jax.lax.dot_general(
        ht[...], eye[...], (((0,), (0,)), ((), ())),
        preferred_element_type=jnp.float32)


def _tc_transpose(ht):
    np_ = ht.shape[1]
    grid = np_ // NBLK
    eye = jnp.eye(HID, dtype=jnp.float32)
    return pl.pallas_call(
        _transpose_body,
        grid=(grid,),
        in_specs=[
            pl.BlockSpec((HID, NBLK), lambda i: (0, i)),
            pl.BlockSpec((HID, HID), lambda i: (0, 0)),
        ],
        out_specs=pl.BlockSpec((NBLK, HID), lambda i: (i, 0)),
        out_shape=jax.ShapeDtypeStruct((np_, HID), jnp.float32),
    )(ht, eye)


@jax.jit
def kernel(pos, edge_index, batch, W1a, b1a, W1b, b1b, W2a, b2a, W2b, b2b):
    del batch
    src = edge_index[0]
    dst = edge_index[1]
    n = pos.shape[0]
    e = src.shape[0]
    np_ = ((n + 1 + NBLK - 1) // NBLK) * NBLK
    lcm = math.lcm(2 * SCG, 2 * SCC, EBLK)
    ep = ((e + lcm - 1) // lcm) * lcm

    post = jnp.zeros((3, np_), jnp.float32).at[:, :n].set(pos.T)
    dst_pad = jnp.concatenate([dst, jnp.full((ep - e,), n, jnp.int32)])
    src_pad = jnp.concatenate([src, jnp.zeros((ep - e,), jnp.int32)])

    wg1t = (W1a[0:3] + W1a[3:6]).T
    wd1t = W1a[3:6].T
    w2h = W2a[0:HID]
    w2pt = W2a[HID:HID + 3].T

    pk1, p2, d2 = _tc_tables(post, wg1t, wd1t, w2pt,
                             b1a.reshape(HID, 1), b2a.reshape(HID, 1))

    u1 = _sc_gather(src_pad, dst_pad, pk1.reshape(HID * np_), ep)
    mt1 = _tc_matmul_t(u1.reshape(HID, ep), W1b, b1b.reshape(HID, 1))
    h1t = _sc_scatter_max(dst_pad, mt1.reshape(HID, ep // 128, 128), np_)
    h1t = h1t.reshape(HID, np_)

    pk2 = _tc_g2(h1t, w2h, p2, d2)
    u2 = _sc_gather(src_pad, dst_pad, pk2.reshape(HID * np_), ep)
    mt2 = _tc_matmul_t(u2.reshape(HID, ep), W2b, b2b.reshape(HID, 1))
    h2t = _sc_scatter_max(dst_pad, mt2.reshape(HID, ep // 128, 128), np_)
    h2t = h2t.reshape(HID, np_)

    return _tc_transpose(h2t)[:n]


# speedup vs baseline: 2.5268x; 2.5268x over previous
import math

import jax
import jax.numpy as jnp
from jax import lax
from jax.experimental import pallas as pl
from jax.experimental.pallas import tpu as pltpu
from jax.experimental.pallas import tpu_sc as plsc

HID = 32
NW = 32
SCG = 2048
SCC = 4096
NBLK = 1024
EBLK = 7168


def _wid():
    return lax.axis_index("s") * 2 + lax.axis_index("c")


def _pack_gd(g, d):
    gu = lax.bitcast_convert_type(g.astype(jnp.bfloat16),
                                  jnp.uint16).astype(jnp.int32)
    du = lax.bitcast_convert_type(d.astype(jnp.bfloat16),
                                  jnp.uint16).astype(jnp.int32)
    return jnp.bitwise_or(gu, jnp.left_shift(du, 16))


def _tables_body(post, wg1t, wd1t, w2pt, b1a, b2a, pk1, p2, d2):
    pt = post[...]
    gv = jax.lax.dot_general(wg1t[...], pt, (((1,), (0,)), ((), ())),
                             preferred_element_type=jnp.float32) + b1a[...]
    dv = jax.lax.dot_general(wd1t[...], pt, (((1,), (0,)), ((), ())),
                             preferred_element_type=jnp.float32)
    d2v = jax.lax.dot_general(w2pt[...], pt, (((1,), (0,)), ((), ())),
                              preferred_element_type=jnp.float32)
    pk1[...] = _pack_gd(gv, dv)
    p2[...] = d2v + b2a[...]
    d2[...] = d2v


def _tc_tables(post, wg1t, wd1t, w2pt, b1a, b2a):
    np_ = post.shape[1]
    grid = np_ // NBLK
    return pl.pallas_call(
        _tables_body,
        grid=(grid,),
        in_specs=[
            pl.BlockSpec((3, NBLK), lambda i: (0, i)),
            pl.BlockSpec((HID, 3), lambda i: (0, 0)),
            pl.BlockSpec((HID, 3), lambda i: (0, 0)),
            pl.BlockSpec((HID, 3), lambda i: (0, 0)),
            pl.BlockSpec((HID, 1), lambda i: (0, 0)),
            pl.BlockSpec((HID, 1), lambda i: (0, 0)),
        ],
        out_specs=[
            pl.BlockSpec((HID, NBLK), lambda i: (0, i)),
            pl.BlockSpec((HID, NBLK), lambda i: (0, i)),
            pl.BlockSpec((HID, NBLK), lambda i: (0, i)),
        ],
        out_shape=[
            jax.ShapeDtypeStruct((HID, np_), jnp.int32),
            jax.ShapeDtypeStruct((HID, np_), jnp.float32),
            jax.ShapeDtypeStruct((HID, np_), jnp.float32),
        ],
    )(post, wg1t, wd1t, w2pt, b1a, b2a)


def _gather_body(src_hbm, dst_hbm, pk_hbm, u_hbm,
                 table, sa, da, sb, db, ua, ub,
                 sema, semb, usema, usemb):
    ep = src_hbm.shape[0]
    np_ = table.shape[0]
    feat = _wid()
    pltpu.sync_copy(pk_hbm.at[pl.ds(feat * np_, np_)], table)
    nc = ep // SCG
    urows = SCG // 128
    mask_hi = jnp.int32(-65536)

    def compute(sv, dv, uv):
        def grp(i, c):
            s = sv[pl.ds(i * 16, 16)]
            d = dv[pl.ds(i * 16, 16)]
            ws = plsc.load_gather(table, [s])
            wd = plsc.load_gather(table, [d])
            g = lax.bitcast_convert_type(jnp.left_shift(ws, 16),
                                         jnp.float32)
            dd = lax.bitcast_convert_type(jnp.bitwise_and(wd, mask_hi),
                                          jnp.float32)
            uv[i // 8, pl.ds((i % 8) * 16, 16)] = jnp.maximum(g - dd, 0.0)
            return c

        lax.fori_loop(0, SCG // 16, grp, 0, unroll=8)

    def issue(ci, sv, dv, sem):
        pltpu.async_copy(src_hbm.at[pl.ds(ci * SCG, SCG)], sv, sem)
        pltpu.async_copy(dst_hbm.at[pl.ds(ci * SCG, SCG)], dv, sem)

    def wait_in(sv, dv, sem):
        pltpu.make_async_copy(src_hbm.at[pl.ds(0, SCG)], sv, sem).wait()
        pltpu.make_async_copy(dst_hbm.at[pl.ds(0, SCG)], dv, sem).wait()

    def drain_u(uv, sem):
        pltpu.make_async_copy(uv, u_hbm.at[feat, pl.ds(0, urows)],
                              sem).wait()

    issue(0, sa, da, sema)

    def pair(k2, carry):
        c0 = 2 * k2
        issue(c0 + 1, sb, db, semb)
        wait_in(sa, da, sema)

        @pl.when(k2 > 0)
        def _():
            drain_u(ua, usema)

        compute(sa, da, ua)
        pltpu.async_copy(ua, u_hbm.at[feat, pl.ds(c0 * urows, urows)],
                         usema)

        @pl.when(c0 + 2 < nc)
        def _():
            issue(c0 + 2, sa, da, sema)

        wait_in(sb, db, semb)

        @pl.when(k2 > 0)
        def _():
            drain_u(ub, usemb)

        compute(sb, db, ub)
        pltpu.async_copy(ub, u_hbm.at[feat, pl.ds((c0 + 1) * urows,
                                                  urows)], usemb)
        return carry

    lax.fori_loop(0, nc // 2, pair, 0)
    drain_u(ua, usema)
    drain_u(ub, usemb)


def _sc_gather(src, dst, pk_flat, ep):
    mesh = plsc.VectorSubcoreMesh(core_axis_name="c", subcore_axis_name="s")
    f = pl.kernel(
        _gather_body,
        out_type=jax.ShapeDtypeStruct((HID, ep // 128, 128), jnp.float32),
        mesh=mesh,
        compiler_params=pltpu.CompilerParams(needs_layout_passes=False),
        scratch_types=[
            pltpu.VMEM((pk_flat.shape[0] // HID,), jnp.int32),
            pltpu.VMEM((SCG,), jnp.int32),
            pltpu.VMEM((SCG,), jnp.int32),
            pltpu.VMEM((SCG,), jnp.int32),
            pltpu.VMEM((SCG,), jnp.int32),
            pltpu.VMEM((SCG // 128, 128), jnp.float32),
            pltpu.VMEM((SCG // 128, 128), jnp.float32),
            pltpu.SemaphoreType.DMA,
            pltpu.SemaphoreType.DMA,
            pltpu.SemaphoreType.DMA,
            pltpu.SemaphoreType.DMA,
        ],
    )
    return f(src, dst, pk_flat)


def _matmul_t_body(u, wb, bbt, mt):
    mt[...] = jax.lax.dot_general(
        wb[...], u[...], (((0,), (0,)), ((), ())),
        preferred_element_type=jnp.float32) + bbt[...]


def _tc_matmul_t(ut, wb, bbt):
    e = ut.shape[1]
    grid = e // EBLK
    return pl.pallas_call(
        _matmul_t_body,
        grid=(grid,),
        in_specs=[
            pl.BlockSpec((HID, EBLK), lambda i: (0, i)),
            pl.BlockSpec((HID, HID), lambda i: (0, 0)),
            pl.BlockSpec((HID, 1), lambda i: (0, 0)),
        ],
        out_specs=pl.BlockSpec((HID, EBLK), lambda i: (0, i)),
        out_shape=jax.ShapeDtypeStruct((HID, e), jnp.float32),
    )(ut, wb, bbt)


def _scatter_body(dst_hbm, mt_hbm, out_hbm, dsta, ma, dstb, mb,
                  sema, semb, table):
    ep = dst_hbm.shape[0]
    np_ = table.shape[0]
    feat = _wid()
    nrows = SCC // 128
    nc = ep // SCC

    zeros = jnp.zeros((16,), jnp.float32)

    def zbody(i, carry):
        table[pl.ds(i * 16, 16)] = zeros
        return carry

    lax.fori_loop(0, np_ // 16, zbody, 0, unroll=8)

    def compute(dst_v, m_v):
        def grp(r, c2):
            unsat_any = jnp.zeros((16,), jnp.bool_)
            for h in range(8):
                d = dst_v[pl.ds(r * 128 + h * 16, 16)]
                v = m_v[r, pl.ds(h * 16, 16)]
                cur = plsc.load_gather(table, [d])
                plsc.store_scatter(table, [d], jnp.maximum(cur, v))
                chk = plsc.load_gather(table, [d])
                unsat_any = jnp.logical_or(unsat_any, chk < v)

            @pl.when(jnp.any(unsat_any))
            def _retry():
                for h in range(8):
                    d = dst_v[pl.ds(r * 128 + h * 16, 16)]
                    v = m_v[r, pl.ds(h * 16, 16)]
                    chk = plsc.load_gather(table, [d])

                    def cond(mask):
                        return jnp.any(mask)

                    def body(mask):
                        cur2 = plsc.load_gather(table, [d], mask=mask)
                        plsc.store_scatter(table, [d],
                                           jnp.maximum(cur2, v), mask=mask)
                        chk2 = plsc.load_gather(table, [d], mask=mask)
                        return jnp.logical_and(mask, chk2 < v)

                    lax.while_loop(cond, body, chk < v)

            return c2

        lax.fori_loop(0, nrows, grp, 0)

    def issue(ci, dst_v, m_v, sem):
        pltpu.async_copy(dst_hbm.at[pl.ds(ci * SCC, SCC)], dst_v, sem)
        pltpu.async_copy(mt_hbm.at[feat, pl.ds(ci * nrows, nrows)], m_v,
                         sem)

    def wait_in(dst_v, m_v, sem):
        pltpu.make_async_copy(dst_hbm.at[pl.ds(0, SCC)], dst_v, sem).wait()
        pltpu.make_async_copy(mt_hbm.at[feat, pl.ds(0, nrows)], m_v,
                              sem).wait()

    issue(0, dsta, ma, sema)

    def pair(k2, carry):
        c0 = 2 * k2
        issue(c0 + 1, dstb, mb, semb)
        wait_in(dsta, ma, sema)
        compute(dsta, ma)

        @pl.when(c0 + 2 < nc)
        def _():
            issue(c0 + 2, dsta, ma, sema)

        wait_in(dstb, mb, semb)
        compute(dstb, mb)
        return carry

    lax.fori_loop(0, nc // 2, pair, 0)
    pltpu.sync_copy(table, out_hbm.at[pl.ds(feat * np_, np_)])


def _sc_scatter_max(dst_pad, mt3, np_):
    mesh = plsc.VectorSubcoreMesh(core_axis_name="c", subcore_axis_name="s")
    f = pl.kernel(
        _scatter_body,
        out_type=jax.ShapeDtypeStruct((HID * np_,), jnp.float32),
        mesh=mesh,
        compiler_params=pltpu.CompilerParams(needs_layout_passes=False),
        scratch_types=[
            pltpu.VMEM((SCC,), jnp.int32),
            pltpu.VMEM((SCC // 128, 128), jnp.float32),
            pltpu.VMEM((SCC,), jnp.int32),
            pltpu.VMEM((SCC // 128, 128), jnp.float32),
            pltpu.SemaphoreType.DMA,
            pltpu.SemaphoreType.DMA,
            pltpu.VMEM((np_,), jnp.float32),
        ],
    )
    return f(dst_pad, mt3)


def _g2_body(h1t, w2h, p2, d2, pk2):
    g2v = jax.lax.dot_general(
        w2h[...], h1t[...], (((0,), (0,)), ((), ())),
        preferred_element_type=jnp.float32) + p2[...]
    pk2[...] = _pack_gd(g2v, d2[...])


def _tc_g2(h1t, w2h, p2, d2):
    np_ = h1t.shape[1]
    grid = np_ // NBLK
    return pl.pallas_call(
        _g2_body,
        grid=(grid,),
        in_specs=[
            pl.BlockSpec((HID, NBLK), lambda i: (0, i)),
            pl.BlockSpec((HID, HID), lambda i: (0, 0)),
            pl.BlockSpec((HID, NBLK), lambda i: (0, i)),
            pl.BlockSpec((HID, NBLK), lambda i: (0, i)),
        ],
        out_specs=pl.BlockSpec((HID, NBLK), lambda i: (0, i)),
        out_shape=jax.ShapeDtypeStruct((HID, np_), jnp.int32),
    )(h1t, w2h, p2, d2)


def _transpose_body(ht, eye, out):
    out[...] = jax.lax.dot_general(
        ht[...], eye[...], (((0,), (0,)), ((), ())),
        preferred_element_type=jnp.float32)


def _tc_transpose(ht):
    np_ = ht.shape[1]
    grid = np_ // NBLK
    eye = jnp.eye(HID, dtype=jnp.float32)
    return pl.pallas_call(
        _transpose_body,
        grid=(grid,),
        in_specs=[
            pl.BlockSpec((HID, NBLK), lambda i: (0, i)),
            pl.BlockSpec((HID, HID), lambda i: (0, 0)),
        ],
        out_specs=pl.BlockSpec((NBLK, HID), lambda i: (i, 0)),
        out_shape=jax.ShapeDtypeStruct((np_, HID), jnp.float32),
    )(ht, eye)


@jax.jit
def kernel(pos, edge_index, batch, W1a, b1a, W1b, b1b, W2a, b2a, W2b, b2b):
    del batch
    src = edge_index[0]
    dst = edge_index[1]
    n = pos.shape[0]
    e = src.shape[0]
    np_ = ((n + 1 + NBLK - 1) // NBLK) * NBLK
    lcm = math.lcm(2 * SCG, 2 * SCC, EBLK)
    ep = ((e + lcm - 1) // lcm) * lcm

    post = jnp.zeros((3, np_), jnp.float32).at[:, :n].set(pos.T)
    dst_pad = jnp.concatenate([dst, jnp.full((ep - e,), n, jnp.int32)])
    src_pad = jnp.concatenate([src, jnp.zeros((ep - e,), jnp.int32)])

    wg1t = (W1a[0:3] + W1a[3:6]).T
    wd1t = W1a[3:6].T
    w2h = W2a[0:HID]
    w2pt = W2a[HID:HID + 3].T

    pk1, p2, d2 = _tc_tables(post, wg1t, wd1t, w2pt,
                             b1a.reshape(HID, 1), b2a.reshape(HID, 1))

    u1 = _sc_gather(src_pad, dst_pad, pk1.reshape(HID * np_), ep)
    mt1 = _tc_matmul_t(u1.reshape(HID, ep), W1b, b1b.reshape(HID, 1))

    h1t = _sc_scatter_max(dst_pad, mt1.reshape(HID, ep // 128, 128), np_)
    h1t = h1t.reshape(HID, np_)

    pk2 = _tc_g2(h1t, w2h, p2, d2)
    u2 = _sc_gather(src_pad, dst_pad, pk2.reshape(HID * np_), ep)
    mt2 = _tc_matmul_t(u2.reshape(HID, ep), W2b, b2b.reshape(HID, 1))
    h2t = _sc_scatter_max(dst_pad, mt2.reshape(HID, ep // 128, 128), np_)
    h2t = h2t.reshape(HID, np_)

    return _tc_transpose(h2t)[:n]


# speedup vs baseline: 2.5292x; 1.0010x over previous
import math

import jax
import jax.numpy as jnp
from jax import lax
from jax.experimental import pallas as pl
from jax.experimental.pallas import tpu as pltpu
from jax.experimental.pallas import tpu_sc as plsc

HID = 32
NW = 32
SCG = 4096
SCC = 4096
NBLK = 1024
EBLK = 7168


def _wid():
    return lax.axis_index("s") * 2 + lax.axis_index("c")


def _pack_gd(g, d):
    gu = lax.bitcast_convert_type(g.astype(jnp.bfloat16),
                                  jnp.uint16).astype(jnp.int32)
    du = lax.bitcast_convert_type(d.astype(jnp.bfloat16),
                                  jnp.uint16).astype(jnp.int32)
    return jnp.bitwise_or(gu, jnp.left_shift(du, 16))


def _tables_body(post, wg1t, wd1t, w2pt, b1a, b2a, pk1, p2, d2):
    pt = post[...]
    gv = jax.lax.dot_general(wg1t[...], pt, (((1,), (0,)), ((), ())),
                             preferred_element_type=jnp.float32) + b1a[...]
    dv = jax.lax.dot_general(wd1t[...], pt, (((1,), (0,)), ((), ())),
                             preferred_element_type=jnp.float32)
    d2v = jax.lax.dot_general(w2pt[...], pt, (((1,), (0,)), ((), ())),
                              preferred_element_type=jnp.float32)
    pk1[...] = _pack_gd(gv, dv)
    p2[...] = d2v + b2a[...]
    d2[...] = d2v


def _tc_tables(post, wg1t, wd1t, w2pt, b1a, b2a):
    np_ = post.shape[1]
    grid = np_ // NBLK
    return pl.pallas_call(
        _tables_body,
        grid=(grid,),
        in_specs=[
            pl.BlockSpec((3, NBLK), lambda i: (0, i)),
            pl.BlockSpec((HID, 3), lambda i: (0, 0)),
            pl.BlockSpec((HID, 3), lambda i: (0, 0)),
            pl.BlockSpec((HID, 3), lambda i: (0, 0)),
            pl.BlockSpec((HID, 1), lambda i: (0, 0)),
            pl.BlockSpec((HID, 1), lambda i: (0, 0)),
        ],
        out_specs=[
            pl.BlockSpec((HID, NBLK), lambda i: (0, i)),
            pl.BlockSpec((HID, NBLK), lambda i: (0, i)),
            pl.BlockSpec((HID, NBLK), lambda i: (0, i)),
        ],
        out_shape=[
            jax.ShapeDtypeStruct((HID, np_), jnp.int32),
            jax.ShapeDtypeStruct((HID, np_), jnp.float32),
            jax.ShapeDtypeStruct((HID, np_), jnp.float32),
        ],
    )(post, wg1t, wd1t, w2pt, b1a, b2a)


def _gather_body(src_hbm, dst_hbm, pk_hbm, u_hbm,
                 table, sa, da, sb, db, ua, ub,
                 sema, semb, usema, usemb):
    ep = src_hbm.shape[0]
    np_ = table.shape[0]
    feat = _wid()
    pltpu.sync_copy(pk_hbm.at[pl.ds(feat * np_, np_)], table)
    nc = ep // SCG
    urows = SCG // 128
    mask_hi = jnp.int32(-65536)

    def compute(sv, dv, uv):
        def grp(i, c):
            s = sv[pl.ds(i * 16, 16)]
            d = dv[pl.ds(i * 16, 16)]
            ws = plsc.load_gather(table, [s])
            wd = plsc.load_gather(table, [d])
            g = lax.bitcast_convert_type(jnp.left_shift(ws, 16),
                                         jnp.float32)
            dd = lax.bitcast_convert_type(jnp.bitwise_and(wd, mask_hi),
                                          jnp.float32)
            uv[i // 8, pl.ds((i % 8) * 16, 16)] = jnp.maximum(g - dd, 0.0)
            return c

        lax.fori_loop(0, SCG // 16, grp, 0, unroll=8)

    def issue(ci, sv, dv, sem):
        pltpu.async_copy(src_hbm.at[pl.ds(ci * SCG, SCG)], sv, sem)
        pltpu.async_copy(dst_hbm.at[pl.ds(ci * SCG, SCG)], dv, sem)

    def wait_in(sv, dv, sem):
        pltpu.make_async_copy(src_hbm.at[pl.ds(0, SCG)], sv, sem).wait()
        pltpu.make_async_copy(dst_hbm.at[pl.ds(0, SCG)], dv, sem).wait()

    def drain_u(uv, sem):
        pltpu.make_async_copy(uv, u_hbm.at[feat, pl.ds(0, urows)],
                              sem).wait()

    issue(0, sa, da, sema)

    def pair(k2, carry):
        c0 = 2 * k2
        issue(c0 + 1, sb, db, semb)
        wait_in(sa, da, sema)

        @pl.when(k2 > 0)
        def _():
            drain_u(ua, usema)

        compute(sa, da, ua)
        pltpu.async_copy(ua, u_hbm.at[feat, pl.ds(c0 * urows, urows)],
                         usema)

        @pl.when(c0 + 2 < nc)
        def _():
            issue(c0 + 2, sa, da, sema)

        wait_in(sb, db, semb)

        @pl.when(k2 > 0)
        def _():
            drain_u(ub, usemb)

        compute(sb, db, ub)
        pltpu.async_copy(ub, u_hbm.at[feat, pl.ds((c0 + 1) * urows,
                                                  urows)], usemb)
        return carry

    lax.fori_loop(0, nc // 2, pair, 0)
    drain_u(ua, usema)
    drain_u(ub, usemb)


def _sc_gather(src, dst, pk_flat, ep):
    mesh = plsc.VectorSubcoreMesh(core_axis_name="c", subcore_axis_name="s")
    f = pl.kernel(
        _gather_body,
        out_type=jax.ShapeDtypeStruct((HID, ep // 128, 128), jnp.float32),
        mesh=mesh,
        compiler_params=pltpu.CompilerParams(needs_layout_passes=False),
        scratch_types=[
            pltpu.VMEM((pk_flat.shape[0] // HID,), jnp.int32),
            pltpu.VMEM((SCG,), jnp.int32),
            pltpu.VMEM((SCG,), jnp.int32),
            pltpu.VMEM((SCG,), jnp.int32),
            pltpu.VMEM((SCG,), jnp.int32),
            pltpu.VMEM((SCG // 128, 128), jnp.float32),
            pltpu.VMEM((SCG // 128, 128), jnp.float32),
            pltpu.SemaphoreType.DMA,
            pltpu.SemaphoreType.DMA,
            pltpu.SemaphoreType.DMA,
            pltpu.SemaphoreType.DMA,
        ],
    )
    return f(src, dst, pk_flat)


def _matmul_t_body(u, wb, bbt, mt):
    mt[...] = jax.lax.dot_general(
        wb[...], u[...], (((0,), (0,)), ((), ())),
        preferred_element_type=jnp.float32) + bbt[...]


def _tc_matmul_t(ut, wb, bbt):
    e = ut.shape[1]
    grid = e // EBLK
    return pl.pallas_call(
        _matmul_t_body,
        grid=(grid,),
        in_specs=[
            pl.BlockSpec((HID, EBLK), lambda i: (0, i)),
            pl.BlockSpec((HID, HID), lambda i: (0, 0)),
            pl.BlockSpec((HID, 1), lambda i: (0, 0)),
        ],
        out_specs=pl.BlockSpec((HID, EBLK), lambda i: (0, i)),
        out_shape=jax.ShapeDtypeStruct((HID, e), jnp.float32),
    )(ut, wb, bbt)


def _scatter_body(dst_hbm, mt_hbm, out_hbm, dsta, ma, dstb, mb,
                  sema, semb, table):
    ep = dst_hbm.shape[0]
    np_ = table.shape[0]
    feat = _wid()
    nrows = SCC // 128
    nc = ep // SCC

    zeros = jnp.zeros((16,), jnp.float32)

    def zbody(i, carry):
        table[pl.ds(i * 16, 16)] = zeros
        return carry

    lax.fori_loop(0, np_ // 16, zbody, 0, unroll=8)

    def compute(dst_v, m_v):
        def grp(r, c2):
            unsat_any = jnp.zeros((16,), jnp.bool_)
            for h in range(8):
                d = dst_v[pl.ds(r * 128 + h * 16, 16)]
                v = m_v[r, pl.ds(h * 16, 16)]
                cur = plsc.load_gather(table, [d])
                plsc.store_scatter(table, [d], jnp.maximum(cur, v))
                chk = plsc.load_gather(table, [d])
                unsat_any = jnp.logical_or(unsat_any, chk < v)

            @pl.when(jnp.any(unsat_any))
            def _retry():
                for h in range(8):
                    d = dst_v[pl.ds(r * 128 + h * 16, 16)]
                    v = m_v[r, pl.ds(h * 16, 16)]
                    chk = plsc.load_gather(table, [d])

                    def cond(mask):
                        return jnp.any(mask)

                    def body(mask):
                        cur2 = plsc.load_gather(table, [d], mask=mask)
                        plsc.store_scatter(table, [d],
                                           jnp.maximum(cur2, v), mask=mask)
                        chk2 = plsc.load_gather(table, [d], mask=mask)
                        return jnp.logical_and(mask, chk2 < v)

                    lax.while_loop(cond, body, chk < v)

            return c2

        lax.fori_loop(0, nrows, grp, 0)

    def issue(ci, dst_v, m_v, sem):
        pltpu.async_copy(dst_hbm.at[pl.ds(ci * SCC, SCC)], dst_v, sem)
        pltpu.async_copy(mt_hbm.at[feat, pl.ds(ci * nrows, nrows)], m_v,
                         sem)

    def wait_in(dst_v, m_v, sem):
        pltpu.make_async_copy(dst_hbm.at[pl.ds(0, SCC)], dst_v, sem).wait()
        pltpu.make_async_copy(mt_hbm.at[feat, pl.ds(0, nrows)], m_v,
                              sem).wait()

    issue(0, dsta, ma, sema)

    def pair(k2, carry):
        c0 = 2 * k2
        issue(c0 + 1, dstb, mb, semb)
        wait_in(dsta, ma, sema)
        compute(dsta, ma)

        @pl.when(c0 + 2 < nc)
        def _():
            issue(c0 + 2, dsta, ma, sema)

        wait_in(dstb, mb, semb)
        compute(dstb, mb)
        return carry

    lax.fori_loop(0, nc // 2, pair, 0)
    pltpu.sync_copy(table, out_hbm.at[pl.ds(feat * np_, np_)])


def _sc_scatter_max(dst_pad, mt3, np_):
    mesh = plsc.VectorSubcoreMesh(core_axis_name="c", subcore_axis_name="s")
    f = pl.kernel(
        _scatter_body,
        out_type=jax.ShapeDtypeStruct((HID * np_,), jnp.float32),
        mesh=mesh,
        compiler_params=pltpu.CompilerParams(needs_layout_passes=False),
        scratch_types=[
            pltpu.VMEM((SCC,), jnp.int32),
            pltpu.VMEM((SCC // 128, 128), jnp.float32),
            pltpu.VMEM((SCC,), jnp.int32),
            pltpu.VMEM((SCC // 128, 128), jnp.float32),
            pltpu.SemaphoreType.DMA,
            pltpu.SemaphoreType.DMA,
            pltpu.VMEM((np_,), jnp.float32),
        ],
    )
    return f(dst_pad, mt3)


def _g2_body(h1t, w2h, p2, d2, pk2):
    g2v = jax.lax.dot_general(
        w2h[...], h1t[...], (((0,), (0,)), ((), ())),
        preferred_element_type=jnp.float32) + p2[...]
    pk2[...] = _pack_gd(g2v, d2[...])


def _tc_g2(h1t, w2h, p2, d2):
    np_ = h1t.shape[1]
    grid = np_ // NBLK
    return pl.pallas_call(
        _g2_body,
        grid=(grid,),
        in_specs=[
            pl.BlockSpec((HID, NBLK), lambda i: (0, i)),
            pl.BlockSpec((HID, HID), lambda i: (0, 0)),
            pl.BlockSpec((HID, NBLK), lambda i: (0, i)),
            pl.BlockSpec((HID, NBLK), lambda i: (0, i)),
        ],
        out_specs=pl.BlockSpec((HID, NBLK), lambda i: (0, i)),
        out_shape=jax.ShapeDtypeStruct((HID, np_), jnp.int32),
    )(h1t, w2h, p2, d2)


def _transpose_body(ht, eye, out):
    out[...] = jax.lax.dot_general(
        ht[...], eye[...], (((0,), (0,)), ((), ())),
        preferred_element_type=jnp.float32)


def _tc_transpose(ht):
    np_ = ht.shape[1]
    grid = np_ // NBLK
    eye = jnp.eye(HID, dtype=jnp.float32)
    return pl.pallas_call(
        _transpose_body,
        grid=(grid,),
        in_specs=[
            pl.BlockSpec((HID, NBLK), lambda i: (0, i)),
            pl.BlockSpec((HID, HID), lambda i: (0, 0)),
        ],
        out_specs=pl.BlockSpec((NBLK, HID), lambda i: (i, 0)),
        out_shape=jax.ShapeDtypeStruct((np_, HID), jnp.float32),
    )(ht, eye)


@jax.jit
def kernel(pos, edge_index, batch, W1a, b1a, W1b, b1b, W2a, b2a, W2b, b2b):
    del batch
    src = edge_index[0]
    dst = edge_index[1]
    n = pos.shape[0]
    e = src.shape[0]
    np_ = ((n + 1 + NBLK - 1) // NBLK) * NBLK
    lcm = math.lcm(2 * SCG, 2 * SCC, EBLK)
    ep = ((e + lcm - 1) // lcm) * lcm

    post = jnp.zeros((3, np_), jnp.float32).at[:, :n].set(pos.T)
    dst_pad = jnp.concatenate([dst, jnp.full((ep - e,), n, jnp.int32)])
    src_pad = jnp.concatenate([src, jnp.zeros((ep - e,), jnp.int32)])

    wg1t = (W1a[0:3] + W1a[3:6]).T
    wd1t = W1a[3:6].T
    w2h = W2a[0:HID]
    w2pt = W2a[HID:HID + 3].T

    pk1, p2, d2 = _tc_tables(post, wg1t, wd1t, w2pt,
                             b1a.reshape(HID, 1), b2a.reshape(HID, 1))

    u1 = _sc_gather(src_pad, dst_pad, pk1.reshape(HID * np_), ep)
    mt1 = _tc_matmul_t(u1.reshape(HID, ep), W1b, b1b.reshape(HID, 1))
    h1t = _sc_scatter_max(dst_pad, mt1.reshape(HID, ep // 128, 128), np_)
    h1t = h1t.reshape(HID, np_)

    pk2 = _tc_g2(h1t, w2h, p2, d2)
    u2 = _sc_gather(src_pad, dst_pad, pk2.reshape(HID * np_), ep)
    mt2 = _tc_matmul_t(u2.reshape(HID, ep), W2b, b2b.reshape(HID, 1))
    h2t = _sc_scatter_max(dst_pad, mt2.reshape(HID, ep // 128, 128), np_)
    h2t = h2t.reshape(HID, np_)

    return _tc_transpose(h2t)[:n]
